# baseline (speedup 1.0000x reference)
import functools

import jax
import jax.numpy as jnp
from jax import lax
from jax.experimental import pallas as pl
from jax.experimental.pallas import tpu as pltpu
from jax.experimental.pallas import tpu_sc as plsc

B, H, W = 8, 512, 512
ROWS = B * H
N = ROWS * W
HW = H * W
NC, NS = 2, 16
NW = NC * NS
RPW = ROWS // NW
CAP = RPW * W
REG = CAP + 16
NEG = float("-inf")


def _wid():
    return lax.axis_index("s") * NC + lax.axis_index("c")


def _p1_body(heat, sidx_o, sconf_o, counts_o, xraw, xpad, hbuf, idxc, confc, cstage):
    wid = _wid()
    lane = lax.iota(jnp.int32, 16)
    zi = jnp.zeros(16, jnp.int32)
    ninf = jnp.zeros(16, jnp.float32) + NEG

    def pad_init(i, _):
        xpad[pl.ds(i * 520, 16)] = ninf
        xpad[pl.ds(i * 520 + 504, 16)] = ninf
        return 0
    lax.fori_loop(0, 18, pad_init, 0)

    def block(blk, carry):
        off, flushed = carry
        g0 = wid * RPW + blk * 16
        pltpu.sync_copy(heat.at[pl.ds(pl.multiple_of(g0 * W, 512), 16 * W)], xraw.at[pl.ds(W, 16 * W)])
        top_in = lax.rem(g0, H) != 0

        @pl.when(top_in)
        def _():
            pltpu.sync_copy(heat.at[pl.ds(pl.multiple_of((g0 - 1) * W, 512), W)], xraw.at[pl.ds(0, W)])

        @pl.when(jnp.logical_not(top_in))
        def _():
            def z(k, _):
                xraw[pl.ds(k * 16, 16)] = ninf
                return 0
            lax.fori_loop(0, 32, z, 0)

        bot_in = lax.rem(g0 + 16, H) != 0

        @pl.when(bot_in)
        def _():
            pltpu.sync_copy(heat.at[pl.ds(pl.multiple_of((g0 + 16) * W, 512), W)], xraw.at[pl.ds(17 * W, W)])

        @pl.when(jnp.logical_not(bot_in))
        def _():
            def z(k, _):
                xraw[pl.ds(17 * W + k * 16, 16)] = ninf
                return 0
            lax.fori_loop(0, 32, z, 0)

        def pad_row(i, _):
            def pc(k, _):
                xpad[pl.ds(i * 520 + 1 + k * 16, 16)] = xraw[pl.ds(i * W + k * 16, 16)]
                return 0
            lax.fori_loop(0, 32, pc, 0)
            return 0
        lax.fori_loop(0, 18, pad_row, 0)

        def h_row(i, _):
            def hk(k, _):
                c = k * 16
                a = xpad[pl.ds(i * 520 + c, 16)]
                m = xpad[pl.ds(i * 520 + 1 + c, 16)]
                r = xpad[pl.ds(i * 520 + 2 + c, 16)]
                hbuf[pl.ds(i * W + c, 16)] = jnp.maximum(jnp.maximum(a, m), r)
                return 0
            lax.fori_loop(0, 32, hk, 0)
            return 0
        lax.fori_loop(0, 18, h_row, 0)

        def p_row(i, carry):
            off, flushed = carry
            g = g0 + i - 1

            def pk(k, off):
                c = k * 16
                v = jnp.maximum(
                    jnp.maximum(hbuf[pl.ds((i - 1) * W + c, 16)],
                                hbuf[pl.ds(i * W + c, 16)]),
                    hbuf[pl.ds((i + 1) * W + c, 16)])
                x = xpad[pl.ds(i * 520 + 1 + c, 16)]
                kp = (v == x) & (x > 0.5)
                flat = g * W + c + lane
                csum = plsc.cumsum(kp.astype(jnp.int32))
                pos = jnp.maximum(off + csum - 1, 0)
                plsc.store_scatter(idxc, [pos], flat, mask=kp)
                plsc.store_scatter(confc, [pos], x, mask=kp)
                return off + csum[15]

            off = lax.fori_loop(0, 32, pk, off)
            pred = off >= 2048

            @pl.when(pred)
            def _():
                pltpu.sync_copy(idxc.at[pl.ds(0, 2048)],
                                sidx_o.at[pl.ds(pl.multiple_of(wid * REG + flushed, 16), 2048)])
                pltpu.sync_copy(confc.at[pl.ds(0, 2048)],
                                sconf_o.at[pl.ds(pl.multiple_of(wid * REG + flushed, 16), 2048)])

                def mv(t, _):
                    idxc[pl.ds(t * 16, 16)] = idxc[pl.ds(2048 + t * 16, 16)]
                    confc[pl.ds(t * 16, 16)] = confc[pl.ds(2048 + t * 16, 16)]
                    return 0
                lax.fori_loop(0, 33, mv, 0)

            off = jnp.where(pred, off - 2048, off)
            flushed = jnp.where(pred, flushed + 2048, flushed)
            return (off, flushed)

        return lax.fori_loop(1, 17, p_row, (off, flushed))

    off, flushed = lax.fori_loop(0, RPW // 16, block,
                                 (jnp.int32(0), jnp.int32(0)))

    nrem = (off + 15) // 16

    def ff(t, _):
        pltpu.sync_copy(idxc.at[pl.ds(t * 16, 16)],
                        sidx_o.at[pl.ds(pl.multiple_of(wid * REG + flushed + t * 16, 16), 16)])
        pltpu.sync_copy(confc.at[pl.ds(t * 16, 16)],
                        sconf_o.at[pl.ds(pl.multiple_of(wid * REG + flushed + t * 16, 16), 16)])
        return 0
    lax.fori_loop(0, nrem, ff, 0)

    cstage[...] = zi + (flushed + off)
    pltpu.sync_copy(cstage, counts_o.at[pl.ds(pl.multiple_of(wid * 16, 16), 16)])


def _p2_body(heat, sidx_i, sconf_i, counts_i, posflat_o, conf_o, zflag_o,
             cbuf, headsI, headsC, sI, sC, tI, tC, zstage):
    wid = _wid()
    lane = lax.iota(jnp.int32, 16)
    pltpu.sync_copy(counts_i, cbuf)

    def pf(i, carry):
        sw, tot = carry
        ci = cbuf[pl.ds(i * 16, 16)][0]
        sw = sw + jnp.where(i < wid, ci, jnp.int32(0))
        return (sw, tot + ci)
    s_w, K = lax.fori_loop(0, NW, pf, (jnp.int32(0), jnp.int32(0)))
    c_w = cbuf[pl.ds(wid * 16, 16)][0]
    e_w = s_w + c_w

    pltpu.sync_copy(heat.at[pl.ds(0, 16)], tC.at[pl.ds(0, 16)])
    cfv = jnp.zeros(16, jnp.float32) + tC[pl.ds(0, 16)][0]

    def hd(i, _):
        pltpu.sync_copy(sidx_i.at[pl.ds(pl.multiple_of(i * REG, 16), 16)], headsI.at[pl.ds(i * 16, 16)])
        pltpu.sync_copy(sconf_i.at[pl.ds(pl.multiple_of(i * REG, 16), 16)], headsC.at[pl.ds(i * 16, 16)])
        return 0
    lax.fori_loop(wid + 1, NW, hd, 0)

    def nx(i, carry):
        t, mI, mC = carry
        ci = cbuf[pl.ds(i * 16, 16)][0]
        rel = lane - t
        valid = (rel >= 0) & (rel < ci)
        g = jnp.clip(rel, 0, 15)
        hI = headsI[pl.ds(i * 16, 16)]
        hC = headsC[pl.ds(i * 16, 16)]
        mI = jnp.where(valid, hI[g], mI)
        mC = jnp.where(valid, hC[g], mC)
        return (t + ci, mI, mC)
    _, mI, mC = lax.fori_loop(wid + 1, NW, nx,
                              (jnp.int32(0), jnp.zeros(16, jnp.int32), cfv))

    out_start = (s_w + 15) // 16 * 16
    d = out_start - s_w
    out_end = (e_w + 15) // 16 * 16
    nch = (out_end - out_start) // 16
    nblk = (nch + 127) // 128

    def blk_body(blk, _):
        pltpu.sync_copy(sidx_i.at[pl.ds(pl.multiple_of(wid * REG + blk * 2048, 16), 2064)], sI)
        pltpu.sync_copy(sconf_i.at[pl.ds(pl.multiple_of(wid * REG + blk * 2048, 16), 2064)], sC)
        qe = jnp.minimum(nch, (blk + 1) * 128)

        def chunk(q, _):
            oq = d + 16 * q - blk * 2048
            vI = sI[pl.ds(oq, 16)]
            vC = sC[pl.ds(oq, 16)]
            own = (d + 16 * q + lane) < c_w
            r = jnp.clip(d + 16 * q + lane - c_w, 0, 15)
            fI = jnp.where(own, vI, mI[r])
            fC = jnp.where(own, vC, mC[r])
            bb = fI // HW
            rem = fI % HW
            hh = rem // W
            ww = rem % W
            qs = q - blk * 128
            base = qs * 48 + lane * 3
            plsc.store_scatter(tI, [base], bb)
            plsc.store_scatter(tI, [base + 1], hh)
            plsc.store_scatter(tI, [base + 2], ww)
            tC[pl.ds(qs * 16, 16)] = fC
            return 0
        lax.fori_loop(blk * 128, qe, chunk, 0)

        nq = qe - blk * 128
        obase = out_start + blk * 2048
        full = nq == 128

        @pl.when(full)
        def _():
            pltpu.sync_copy(tI, posflat_o.at[pl.ds(pl.multiple_of(obase * 3, 48), 6144)])
            pltpu.sync_copy(tC, conf_o.at[pl.ds(pl.multiple_of(obase, 16), 2048)])

        @pl.when(jnp.logical_not(full))
        def _():
            def sm(qq, _):
                pltpu.sync_copy(tI.at[pl.ds(qq * 48, 48)],
                                posflat_o.at[pl.ds(pl.multiple_of((obase + qq * 16) * 3, 48), 48)])
                pltpu.sync_copy(tC.at[pl.ds(qq * 16, 16)],
                                conf_o.at[pl.ds(pl.multiple_of(obase + qq * 16, 16), 16)])
                return 0
            lax.fori_loop(0, nq, sm, 0)
        return 0
    lax.fori_loop(0, nblk, blk_body, 0)

    def fz(t, _):
        tI[pl.ds(t * 16, 16)] = jnp.zeros(16, jnp.int32)
        return 0
    lax.fori_loop(0, 384, fz, 0)

    def fc(t, _):
        tC[pl.ds(t * 16, 16)] = cfv
        return 0
    lax.fori_loop(0, 128, fc, 0)

    AK = (K + 15) // 16 * 16
    F16 = (N - AK) // 16
    lo = (wid * F16) // NW
    hi = ((wid + 1) * F16) // NW
    base_el = AK + lo * 16
    nfc = hi - lo
    nbig = nfc // 128

    def fb(m, _):
        pltpu.sync_copy(tI, posflat_o.at[pl.ds(pl.multiple_of((base_el + m * 2048) * 3, 48), 6144)])
        pltpu.sync_copy(tC, conf_o.at[pl.ds(pl.multiple_of(base_el + m * 2048, 16), 2048)])
        return 0
    lax.fori_loop(0, nbig, fb, 0)

    def fs(qq, _):
        o = base_el + nbig * 2048 + qq * 16
        pltpu.sync_copy(tI.at[pl.ds(0, 48)], posflat_o.at[pl.ds(pl.multiple_of(o * 3, 48), 48)])
        pltpu.sync_copy(tC.at[pl.ds(0, 16)], conf_o.at[pl.ds(pl.multiple_of(o, 16), 16)])
        return 0
    lax.fori_loop(0, nfc - nbig * 128, fs, 0)

    @pl.when(wid == 0)
    def _():
        zstage[...] = jnp.zeros(16, jnp.int32)
        pltpu.sync_copy(zstage, zflag_o)


def _mesh():
    return plsc.VectorSubcoreMesh(core_axis_name="c", subcore_axis_name="s")


@jax.jit
def _to_position(heat_flat):
    sidx, sconf, counts = pl.kernel(
        _p1_body,
        out_type=(
            jax.ShapeDtypeStruct((NW * REG,), jnp.int32),
            jax.ShapeDtypeStruct((NW * REG,), jnp.float32),
            jax.ShapeDtypeStruct((NW * 16,), jnp.int32),
        ),
        mesh=_mesh(),
        compiler_params=pltpu.CompilerParams(needs_layout_passes=False),
        scratch_types=[
            pltpu.VMEM((18 * W,), jnp.float32),
            pltpu.VMEM((18 * 520,), jnp.float32),
            pltpu.VMEM((18 * W,), jnp.float32),
            pltpu.VMEM((2576,), jnp.int32),
            pltpu.VMEM((2576,), jnp.float32),
            pltpu.VMEM((16,), jnp.int32),
        ],
    )(heat_flat)
    posflat, conf, zflag = pl.kernel(
        _p2_body,
        out_type=(
            jax.ShapeDtypeStruct((3 * N,), jnp.int32),
            jax.ShapeDtypeStruct((N,), jnp.float32),
            jax.ShapeDtypeStruct((16,), jnp.int32),
        ),
        mesh=_mesh(),
        compiler_params=pltpu.CompilerParams(needs_layout_passes=False),
        scratch_types=[
            pltpu.VMEM((NW * 16,), jnp.int32),
            pltpu.VMEM((NW * 16,), jnp.int32),
            pltpu.VMEM((NW * 16,), jnp.float32),
            pltpu.VMEM((2064,), jnp.int32),
            pltpu.VMEM((2064,), jnp.float32),
            pltpu.VMEM((2048 * 3,), jnp.int32),
            pltpu.VMEM((2048,), jnp.float32),
            pltpu.VMEM((16,), jnp.int32),
        ],
    )(heat_flat, sidx, sconf, counts)
    return posflat, conf, zflag


def kernel(heatmap):
    heat_flat = heatmap.reshape(N)
    posflat, conf, zflag = _to_position(heat_flat)
    pos = posflat.reshape(N, 3)
    confidences = conf
    _ = (pos, confidences)
    return zflag[0].reshape(())


# speedup vs baseline: 1.1804x; 1.1804x over previous
import functools

import jax
import jax.numpy as jnp
from jax import lax
from jax.experimental import pallas as pl
from jax.experimental.pallas import tpu as pltpu
from jax.experimental.pallas import tpu_sc as plsc

B, H, W = 8, 512, 512
ROWS = B * H
N = ROWS * W
HW = H * W
NC, NS = 2, 16
NW = NC * NS
RPW = ROWS // NW
CAP = RPW * W
REG = CAP + 16
NEG = float("-inf")


def _wid():
    return lax.axis_index("s") * NC + lax.axis_index("c")


def _p1_body(heat, sidx_o, sconf_o, counts_o, xraw, vrow, idxc, confc, cstage):
    wid = _wid()
    lane = lax.iota(jnp.int32, 16)
    zi = jnp.zeros(16, jnp.int32)
    ninf = jnp.zeros(16, jnp.float32) + NEG

    vrow[pl.ds(0, 16)] = ninf
    vrow[pl.ds(512, 16)] = ninf

    def block(blk, carry):
        off, flushed = carry
        g0 = wid * RPW + blk * 16
        pltpu.sync_copy(heat.at[pl.ds(pl.multiple_of(g0 * W, 512), 16 * W)], xraw.at[pl.ds(W, 16 * W)])
        top_in = lax.rem(g0, H) != 0

        @pl.when(top_in)
        def _():
            pltpu.sync_copy(heat.at[pl.ds(pl.multiple_of((g0 - 1) * W, 512), W)], xraw.at[pl.ds(0, W)])

        @pl.when(jnp.logical_not(top_in))
        def _():
            def z(k, _):
                xraw[pl.ds(k * 16, 16)] = ninf
                return 0
            lax.fori_loop(0, 32, z, 0)

        bot_in = lax.rem(g0 + 16, H) != 0

        @pl.when(bot_in)
        def _():
            pltpu.sync_copy(heat.at[pl.ds(pl.multiple_of((g0 + 16) * W, 512), W)], xraw.at[pl.ds(17 * W, W)])

        @pl.when(jnp.logical_not(bot_in))
        def _():
            def z(k, _):
                xraw[pl.ds(17 * W + k * 16, 16)] = ninf
                return 0
            lax.fori_loop(0, 32, z, 0)

        def p_row(i, carry):
            off, flushed = carry
            g = g0 + i - 1

            def vk(k, _):
                c = k * 16
                vrow[pl.ds(8 + c, 16)] = jnp.maximum(
                    jnp.maximum(xraw[pl.ds((i - 1) * W + c, 16)],
                                xraw[pl.ds(i * W + c, 16)]),
                    xraw[pl.ds((i + 1) * W + c, 16)])
                return 0
            lax.fori_loop(0, 32, vk, 0)

            def pk(k, off):
                c = k * 16
                v = jnp.maximum(
                    jnp.maximum(vrow[pl.ds(7 + c, 16)], vrow[pl.ds(8 + c, 16)]),
                    vrow[pl.ds(9 + c, 16)])
                x = xraw[pl.ds(i * W + c, 16)]
                kp = (v == x) & (x > 0.5)
                flat = g * W + c + lane
                csum = plsc.cumsum(jnp.where(kp, jnp.int32(1), jnp.int32(0)))
                pos = jnp.maximum(off + csum - 1, 0)
                plsc.store_scatter(idxc, [pos], flat, mask=kp)
                plsc.store_scatter(confc, [pos], x, mask=kp)
                pc = plsc.all_reduce_population_count(kp)
                return off + pc[0]

            off = lax.fori_loop(0, 32, pk, off)
            pred = off >= 2048

            @pl.when(pred)
            def _():
                pltpu.sync_copy(idxc.at[pl.ds(0, 2048)],
                                sidx_o.at[pl.ds(pl.multiple_of(wid * REG + flushed, 16), 2048)])
                pltpu.sync_copy(confc.at[pl.ds(0, 2048)],
                                sconf_o.at[pl.ds(pl.multiple_of(wid * REG + flushed, 16), 2048)])

                def mv(t, _):
                    idxc[pl.ds(t * 16, 16)] = idxc[pl.ds(2048 + t * 16, 16)]
                    confc[pl.ds(t * 16, 16)] = confc[pl.ds(2048 + t * 16, 16)]
                    return 0
                lax.fori_loop(0, 33, mv, 0)

            off = jnp.where(pred, off - 2048, off)
            flushed = jnp.where(pred, flushed + 2048, flushed)
            return (off, flushed)

        return lax.fori_loop(1, 17, p_row, (off, flushed))

    off, flushed = lax.fori_loop(0, RPW // 16, block,
                                 (jnp.int32(0), jnp.int32(0)))

    rem = off
    done = jnp.int32(0)
    for sz in (1024, 512, 256, 128, 64, 32, 16):
        pred = rem >= sz

        @pl.when(pred)
        def _(sz=sz, done=done):
            pltpu.sync_copy(idxc.at[pl.ds(done, sz)],
                            sidx_o.at[pl.ds(pl.multiple_of(wid * REG + flushed + done, 16), sz)])
            pltpu.sync_copy(confc.at[pl.ds(done, sz)],
                            sconf_o.at[pl.ds(pl.multiple_of(wid * REG + flushed + done, 16), sz)])
        step = jnp.where(pred, jnp.int32(sz), jnp.int32(0))
        done = done + step
        rem = rem - step

    @pl.when(rem > 0)
    def _():
        pltpu.sync_copy(idxc.at[pl.ds(done, 16)],
                        sidx_o.at[pl.ds(pl.multiple_of(wid * REG + flushed + done, 16), 16)])
        pltpu.sync_copy(confc.at[pl.ds(done, 16)],
                        sconf_o.at[pl.ds(pl.multiple_of(wid * REG + flushed + done, 16), 16)])

    cstage[...] = zi + (flushed + off)
    pltpu.sync_copy(cstage, counts_o.at[pl.ds(pl.multiple_of(wid * 16, 16), 16)])


def _p2_body(heat, sidx_i, sconf_i, counts_i, posflat_o, conf_o, zflag_o,
             cbuf, headsI, headsC, sI, sC, tI, tC, zstage, fsem):
    wid = _wid()
    lane = lax.iota(jnp.int32, 16)
    pltpu.sync_copy(counts_i, cbuf)

    def pf(i, carry):
        sw, tot = carry
        ci = cbuf[pl.ds(i * 16, 16)][0]
        sw = sw + jnp.where(i < wid, ci, jnp.int32(0))
        return (sw, tot + ci)
    s_w, K = lax.fori_loop(0, NW, pf, (jnp.int32(0), jnp.int32(0)))
    c_w = cbuf[pl.ds(wid * 16, 16)][0]
    e_w = s_w + c_w

    pltpu.sync_copy(heat.at[pl.ds(0, 16)], tC.at[pl.ds(0, 16)])
    cfv = jnp.zeros(16, jnp.float32) + tC[pl.ds(0, 16)][0]

    def hd(i, _):
        pltpu.sync_copy(sidx_i.at[pl.ds(pl.multiple_of(i * REG, 16), 16)], headsI.at[pl.ds(i * 16, 16)])
        pltpu.sync_copy(sconf_i.at[pl.ds(pl.multiple_of(i * REG, 16), 16)], headsC.at[pl.ds(i * 16, 16)])
        return 0
    lax.fori_loop(wid + 1, NW, hd, 0)

    def nx(i, carry):
        t, mI, mC = carry
        ci = cbuf[pl.ds(i * 16, 16)][0]
        rel = lane - t
        valid = (rel >= 0) & (rel < ci)
        g = jnp.clip(rel, 0, 15)
        hI = headsI[pl.ds(i * 16, 16)]
        hC = headsC[pl.ds(i * 16, 16)]
        mI = jnp.where(valid, hI[g], mI)
        mC = jnp.where(valid, hC[g], mC)
        return (t + ci, mI, mC)
    _, mI, mC = lax.fori_loop(wid + 1, NW, nx,
                              (jnp.int32(0), jnp.zeros(16, jnp.int32), cfv))

    out_start = (s_w + 15) // 16 * 16
    d = out_start - s_w
    out_end = (e_w + 15) // 16 * 16
    nch = (out_end - out_start) // 16
    nblk = (nch + 127) // 128

    def blk_body(blk, _):
        pltpu.sync_copy(sidx_i.at[pl.ds(pl.multiple_of(wid * REG + blk * 2048, 16), 2064)], sI)
        pltpu.sync_copy(sconf_i.at[pl.ds(pl.multiple_of(wid * REG + blk * 2048, 16), 2064)], sC)
        qe = jnp.minimum(nch, (blk + 1) * 128)

        def chunk(q, _):
            oq = d + 16 * q - blk * 2048
            vI = sI[pl.ds(oq, 16)]
            vC = sC[pl.ds(oq, 16)]
            own = (d + 16 * q + lane) < c_w
            r = jnp.clip(d + 16 * q + lane - c_w, 0, 15)
            fI = jnp.where(own, vI, mI[r])
            fC = jnp.where(own, vC, mC[r])
            bb = fI // HW
            rem = fI % HW
            hh = rem // W
            ww = rem % W
            qs = q - blk * 128
            base = qs * 48 + lane * 3
            plsc.store_scatter(tI, [base], bb)
            plsc.store_scatter(tI, [base + 1], hh)
            plsc.store_scatter(tI, [base + 2], ww)
            tC[pl.ds(qs * 16, 16)] = fC
            return 0
        lax.fori_loop(blk * 128, qe, chunk, 0)

        nq = qe - blk * 128
        obase = out_start + blk * 2048
        full = nq == 128

        @pl.when(full)
        def _():
            pltpu.sync_copy(tI, posflat_o.at[pl.ds(pl.multiple_of(obase * 3, 48), 6144)])
            pltpu.sync_copy(tC, conf_o.at[pl.ds(pl.multiple_of(obase, 16), 2048)])

        @pl.when(jnp.logical_not(full))
        def _():
            remq = nq
            doneq = jnp.int32(0)
            for szq in (64, 32, 16, 8, 4, 2, 1):
                predq = remq >= szq

                @pl.when(predq)
                def _(szq=szq, doneq=doneq):
                    pltpu.sync_copy(
                        tI.at[pl.ds(doneq * 48, szq * 48)],
                        posflat_o.at[pl.ds(pl.multiple_of((obase + doneq * 16) * 3, 48), szq * 48)])
                    pltpu.sync_copy(
                        tC.at[pl.ds(doneq * 16, szq * 16)],
                        conf_o.at[pl.ds(pl.multiple_of(obase + doneq * 16, 16), szq * 16)])
                stepq = jnp.where(predq, jnp.int32(szq), jnp.int32(0))
                doneq = doneq + stepq
                remq = remq - stepq
        return 0
    lax.fori_loop(0, nblk, blk_body, 0)

    def fz(t, _):
        tI[pl.ds(t * 16, 16)] = jnp.zeros(16, jnp.int32)
        return 0
    lax.fori_loop(0, 384, fz, 0)

    def fc(t, _):
        tC[pl.ds(t * 16, 16)] = cfv
        return 0
    lax.fori_loop(0, 128, fc, 0)

    AK = (K + 15) // 16 * 16
    F16 = (N - AK) // 16
    lo = (wid * F16) // NW
    hi = ((wid + 1) * F16) // NW
    base_el = AK + lo * 16
    nfc = hi - lo
    nbig = nfc // 128

    def fb(m, _):
        pltpu.async_copy(
            tI, posflat_o.at[pl.ds(pl.multiple_of((base_el + m * 2048) * 3, 48), 6144)], fsem)
        pltpu.async_copy(
            tC, conf_o.at[pl.ds(pl.multiple_of(base_el + m * 2048, 16), 2048)], fsem)
        return 0
    lax.fori_loop(0, nbig, fb, 0)

    remf = nfc - nbig * 128
    donef = jnp.int32(0)
    for szf in (64, 32, 16, 8, 4, 2, 1):
        predf = remf >= szf

        @pl.when(predf)
        def _(szf=szf, donef=donef):
            o = base_el + nbig * 2048 + donef * 16
            pltpu.sync_copy(tI.at[pl.ds(0, szf * 48)],
                            posflat_o.at[pl.ds(pl.multiple_of(o * 3, 48), szf * 48)])
            pltpu.sync_copy(tC.at[pl.ds(0, szf * 16)],
                            conf_o.at[pl.ds(pl.multiple_of(o, 16), szf * 16)])
        stepf = jnp.where(predf, jnp.int32(szf), jnp.int32(0))
        donef = donef + stepf
        remf = remf - stepf

    def fd(m, _):
        pltpu.make_async_copy(posflat_o.at[pl.ds(0, 6144)], tI, fsem).wait()
        pltpu.make_async_copy(conf_o.at[pl.ds(0, 2048)], tC, fsem).wait()
        return 0
    lax.fori_loop(0, nbig, fd, 0)

    @pl.when(wid == 0)
    def _():
        zstage[...] = jnp.zeros(16, jnp.int32)
        pltpu.sync_copy(zstage, zflag_o)


def _mesh():
    return plsc.VectorSubcoreMesh(core_axis_name="c", subcore_axis_name="s")


@jax.jit
def _to_position(heat_flat):
    sidx, sconf, counts = pl.kernel(
        _p1_body,
        out_type=(
            jax.ShapeDtypeStruct((NW * REG,), jnp.int32),
            jax.ShapeDtypeStruct((NW * REG,), jnp.float32),
            jax.ShapeDtypeStruct((NW * 16,), jnp.int32),
        ),
        mesh=_mesh(),
        compiler_params=pltpu.CompilerParams(needs_layout_passes=False),
        scratch_types=[
            pltpu.VMEM((18 * W,), jnp.float32),
            pltpu.VMEM((528,), jnp.float32),
            pltpu.VMEM((2576,), jnp.int32),
            pltpu.VMEM((2576,), jnp.float32),
            pltpu.VMEM((16,), jnp.int32),
        ],
    )(heat_flat)
    posflat, conf, zflag = pl.kernel(
        _p2_body,
        out_type=(
            jax.ShapeDtypeStruct((3 * N,), jnp.int32),
            jax.ShapeDtypeStruct((N,), jnp.float32),
            jax.ShapeDtypeStruct((16,), jnp.int32),
        ),
        mesh=_mesh(),
        compiler_params=pltpu.CompilerParams(needs_layout_passes=False),
        scratch_types=[
            pltpu.VMEM((NW * 16,), jnp.int32),
            pltpu.VMEM((NW * 16,), jnp.int32),
            pltpu.VMEM((NW * 16,), jnp.float32),
            pltpu.VMEM((2064,), jnp.int32),
            pltpu.VMEM((2064,), jnp.float32),
            pltpu.VMEM((2048 * 3,), jnp.int32),
            pltpu.VMEM((2048,), jnp.float32),
            pltpu.VMEM((16,), jnp.int32),
            pltpu.SemaphoreType.DMA,
        ],
    )(heat_flat, sidx, sconf, counts)
    return posflat, conf, zflag


def kernel(heatmap):
    heat_flat = heatmap.reshape(N)
    posflat, conf, zflag = _to_position(heat_flat)
    pos = posflat.reshape(N, 3)
    confidences = conf
    _ = (pos, confidences)
    return zflag[0].reshape(())


# speedup vs baseline: 1.2892x; 1.0922x over previous
import functools

import jax
import jax.numpy as jnp
from jax import lax
from jax.experimental import pallas as pl
from jax.experimental.pallas import tpu as pltpu
from jax.experimental.pallas import tpu_sc as plsc

B, H, W = 8, 512, 512
ROWS = B * H
N = ROWS * W
HW = H * W
NC, NS = 2, 16
NW = NC * NS
RPW = ROWS // NW
CAP = RPW * W
REG = CAP + 16
NEG = float("-inf")


def _wid():
    return lax.axis_index("s") * NC + lax.axis_index("c")


def _p1_body(heat, sidx_o, sconf_o, counts_o, xraw, vrow, idxc, confc, cstage):
    wid = _wid()
    lane = lax.iota(jnp.int32, 16)
    zi = jnp.zeros(16, jnp.int32)
    ninf = jnp.zeros(16, jnp.float32) + NEG

    vrow[pl.ds(0, 16)] = ninf
    vrow[pl.ds(512, 16)] = ninf

    def block(blk, carry):
        off, flushed = carry
        g0 = wid * RPW + blk * 16
        pltpu.sync_copy(heat.at[pl.ds(pl.multiple_of(g0 * W, 512), 16 * W)], xraw.at[pl.ds(W, 16 * W)])
        top_in = lax.rem(g0, H) != 0

        @pl.when(top_in)
        def _():
            pltpu.sync_copy(heat.at[pl.ds(pl.multiple_of((g0 - 1) * W, 512), W)], xraw.at[pl.ds(0, W)])

        @pl.when(jnp.logical_not(top_in))
        def _():
            def z(k, _):
                xraw[pl.ds(k * 16, 16)] = ninf
                return 0
            lax.fori_loop(0, 32, z, 0)

        bot_in = lax.rem(g0 + 16, H) != 0

        @pl.when(bot_in)
        def _():
            pltpu.sync_copy(heat.at[pl.ds(pl.multiple_of((g0 + 16) * W, 512), W)], xraw.at[pl.ds(17 * W, W)])

        @pl.when(jnp.logical_not(bot_in))
        def _():
            def z(k, _):
                xraw[pl.ds(17 * W + k * 16, 16)] = ninf
                return 0
            lax.fori_loop(0, 32, z, 0)

        def p_row(i, carry):
            off, flushed = carry
            g = g0 + i - 1

            def vk(kk, _):
                for j in range(4):
                    c = kk * 64 + j * 16
                    vrow[pl.ds(8 + c, 16)] = jnp.maximum(
                        jnp.maximum(xraw[pl.ds((i - 1) * W + c, 16)],
                                    xraw[pl.ds(i * W + c, 16)]),
                        xraw[pl.ds((i + 1) * W + c, 16)])
                return 0
            lax.fori_loop(0, 8, vk, 0)

            def pk(kk, off):
                c0 = kk * 32
                c1 = c0 + 16
                v0 = jnp.maximum(
                    jnp.maximum(vrow[pl.ds(7 + c0, 16)], vrow[pl.ds(8 + c0, 16)]),
                    vrow[pl.ds(9 + c0, 16)])
                x0 = xraw[pl.ds(i * W + c0, 16)]
                v1 = jnp.maximum(
                    jnp.maximum(vrow[pl.ds(7 + c1, 16)], vrow[pl.ds(8 + c1, 16)]),
                    vrow[pl.ds(9 + c1, 16)])
                x1 = xraw[pl.ds(i * W + c1, 16)]
                kp0 = (v0 == x0) & (x0 > 0.5)
                kp1 = (v1 == x1) & (x1 > 0.5)
                cs0 = plsc.cumsum(jnp.where(kp0, jnp.int32(1), jnp.int32(0)))
                cs1 = plsc.cumsum(jnp.where(kp1, jnp.int32(1), jnp.int32(0)))
                pc0 = plsc.all_reduce_population_count(kp0)
                pc1 = plsc.all_reduce_population_count(kp1)
                pos0 = jnp.maximum(off + cs0 - 1, 0)
                plsc.store_scatter(idxc, [pos0], g * W + c0 + lane, mask=kp0)
                plsc.store_scatter(confc, [pos0], x0, mask=kp0)
                off1 = off + pc0[0]
                pos1 = jnp.maximum(off1 + cs1 - 1, 0)
                plsc.store_scatter(idxc, [pos1], g * W + c1 + lane, mask=kp1)
                plsc.store_scatter(confc, [pos1], x1, mask=kp1)
                return off1 + pc1[0]

            off = lax.fori_loop(0, 16, pk, off)
            pred = off >= 2048

            @pl.when(pred)
            def _():
                pltpu.sync_copy(idxc.at[pl.ds(0, 2048)],
                                sidx_o.at[pl.ds(pl.multiple_of(wid * REG + flushed, 16), 2048)])
                pltpu.sync_copy(confc.at[pl.ds(0, 2048)],
                                sconf_o.at[pl.ds(pl.multiple_of(wid * REG + flushed, 16), 2048)])

                def mv(t, _):
                    idxc[pl.ds(t * 16, 16)] = idxc[pl.ds(2048 + t * 16, 16)]
                    confc[pl.ds(t * 16, 16)] = confc[pl.ds(2048 + t * 16, 16)]
                    return 0
                lax.fori_loop(0, 33, mv, 0)

            off = jnp.where(pred, off - 2048, off)
            flushed = jnp.where(pred, flushed + 2048, flushed)
            return (off, flushed)

        return lax.fori_loop(1, 17, p_row, (off, flushed))

    off, flushed = lax.fori_loop(0, RPW // 16, block,
                                 (jnp.int32(0), jnp.int32(0)))

    rem = off
    done = jnp.int32(0)
    for sz in (1024, 512, 256, 128, 64, 32, 16):
        pred = rem >= sz

        @pl.when(pred)
        def _(sz=sz, done=done):
            pltpu.sync_copy(idxc.at[pl.ds(done, sz)],
                            sidx_o.at[pl.ds(pl.multiple_of(wid * REG + flushed + done, 16), sz)])
            pltpu.sync_copy(confc.at[pl.ds(done, sz)],
                            sconf_o.at[pl.ds(pl.multiple_of(wid * REG + flushed + done, 16), sz)])
        step = jnp.where(pred, jnp.int32(sz), jnp.int32(0))
        done = done + step
        rem = rem - step

    @pl.when(rem > 0)
    def _():
        pltpu.sync_copy(idxc.at[pl.ds(done, 16)],
                        sidx_o.at[pl.ds(pl.multiple_of(wid * REG + flushed + done, 16), 16)])
        pltpu.sync_copy(confc.at[pl.ds(done, 16)],
                        sconf_o.at[pl.ds(pl.multiple_of(wid * REG + flushed + done, 16), 16)])

    cstage[...] = zi + (flushed + off)
    pltpu.sync_copy(cstage, counts_o.at[pl.ds(pl.multiple_of(wid * 16, 16), 16)])


def _p2_body(heat, sidx_i, sconf_i, counts_i, posflat_o, conf_o, zflag_o,
             cbuf, headsI, headsC, sI, sC, tI, tC, zstage, fsem):
    wid = _wid()
    lane = lax.iota(jnp.int32, 16)
    pltpu.sync_copy(counts_i, cbuf)

    def pf(i, carry):
        sw, tot = carry
        ci = cbuf[pl.ds(i * 16, 16)][0]
        sw = sw + jnp.where(i < wid, ci, jnp.int32(0))
        return (sw, tot + ci)
    s_w, K = lax.fori_loop(0, NW, pf, (jnp.int32(0), jnp.int32(0)))
    c_w = cbuf[pl.ds(wid * 16, 16)][0]
    e_w = s_w + c_w

    pltpu.sync_copy(heat.at[pl.ds(0, 16)], tC.at[pl.ds(0, 16)])
    cfv = jnp.zeros(16, jnp.float32) + tC[pl.ds(0, 16)][0]

    def hd(i, _):
        pltpu.sync_copy(sidx_i.at[pl.ds(pl.multiple_of(i * REG, 16), 16)], headsI.at[pl.ds(i * 16, 16)])
        pltpu.sync_copy(sconf_i.at[pl.ds(pl.multiple_of(i * REG, 16), 16)], headsC.at[pl.ds(i * 16, 16)])
        return 0
    lax.fori_loop(wid + 1, NW, hd, 0)

    def nx(i, carry):
        t, mI, mC = carry
        ci = cbuf[pl.ds(i * 16, 16)][0]
        rel = lane - t
        valid = (rel >= 0) & (rel < ci)
        g = jnp.clip(rel, 0, 15)
        hI = headsI[pl.ds(i * 16, 16)]
        hC = headsC[pl.ds(i * 16, 16)]
        mI = jnp.where(valid, hI[g], mI)
        mC = jnp.where(valid, hC[g], mC)
        return (t + ci, mI, mC)
    _, mI, mC = lax.fori_loop(wid + 1, NW, nx,
                              (jnp.int32(0), jnp.zeros(16, jnp.int32), cfv))

    out_start = (s_w + 15) // 16 * 16
    d = out_start - s_w
    out_end = (e_w + 15) // 16 * 16
    nch = (out_end - out_start) // 16
    nblk = (nch + 127) // 128

    def blk_body(blk, _):
        pltpu.sync_copy(sidx_i.at[pl.ds(pl.multiple_of(wid * REG + blk * 2048, 16), 2064)], sI)
        pltpu.sync_copy(sconf_i.at[pl.ds(pl.multiple_of(wid * REG + blk * 2048, 16), 2064)], sC)
        qe = jnp.minimum(nch, (blk + 1) * 128)

        def chunk(q, _):
            oq = d + 16 * q - blk * 2048
            vI = sI[pl.ds(oq, 16)]
            vC = sC[pl.ds(oq, 16)]
            own = (d + 16 * q + lane) < c_w
            r = jnp.clip(d + 16 * q + lane - c_w, 0, 15)
            fI = jnp.where(own, vI, mI[r])
            fC = jnp.where(own, vC, mC[r])
            bb = fI // HW
            rem = fI % HW
            hh = rem // W
            ww = rem % W
            qs = q - blk * 128
            base = qs * 48 + lane * 3
            plsc.store_scatter(tI, [base], bb)
            plsc.store_scatter(tI, [base + 1], hh)
            plsc.store_scatter(tI, [base + 2], ww)
            tC[pl.ds(qs * 16, 16)] = fC
            return 0
        lax.fori_loop(blk * 128, qe, chunk, 0)

        nq = qe - blk * 128
        obase = out_start + blk * 2048
        full = nq == 128

        @pl.when(full)
        def _():
            pltpu.sync_copy(tI, posflat_o.at[pl.ds(pl.multiple_of(obase * 3, 48), 6144)])
            pltpu.sync_copy(tC, conf_o.at[pl.ds(pl.multiple_of(obase, 16), 2048)])

        @pl.when(jnp.logical_not(full))
        def _():
            remq = nq
            doneq = jnp.int32(0)
            for szq in (64, 32, 16, 8, 4, 2, 1):
                predq = remq >= szq

                @pl.when(predq)
                def _(szq=szq, doneq=doneq):
                    pltpu.sync_copy(
                        tI.at[pl.ds(doneq * 48, szq * 48)],
                        posflat_o.at[pl.ds(pl.multiple_of((obase + doneq * 16) * 3, 48), szq * 48)])
                    pltpu.sync_copy(
                        tC.at[pl.ds(doneq * 16, szq * 16)],
                        conf_o.at[pl.ds(pl.multiple_of(obase + doneq * 16, 16), szq * 16)])
                stepq = jnp.where(predq, jnp.int32(szq), jnp.int32(0))
                doneq = doneq + stepq
                remq = remq - stepq
        return 0
    lax.fori_loop(0, nblk, blk_body, 0)

    def fz(t, _):
        tI[pl.ds(t * 16, 16)] = jnp.zeros(16, jnp.int32)
        return 0
    lax.fori_loop(0, 384, fz, 0)

    def fc(t, _):
        tC[pl.ds(t * 16, 16)] = cfv
        return 0
    lax.fori_loop(0, 128, fc, 0)

    AK = (K + 15) // 16 * 16
    F16 = (N - AK) // 16
    lo = (wid * F16) // NW
    hi = ((wid + 1) * F16) // NW
    base_el = AK + lo * 16
    nfc = hi - lo
    nbig = nfc // 128

    def fb(m, _):
        pltpu.async_copy(
            tI, posflat_o.at[pl.ds(pl.multiple_of((base_el + m * 2048) * 3, 48), 6144)], fsem)
        pltpu.async_copy(
            tC, conf_o.at[pl.ds(pl.multiple_of(base_el + m * 2048, 16), 2048)], fsem)
        return 0
    lax.fori_loop(0, nbig, fb, 0)

    remf = nfc - nbig * 128
    donef = jnp.int32(0)
    for szf in (64, 32, 16, 8, 4, 2, 1):
        predf = remf >= szf

        @pl.when(predf)
        def _(szf=szf, donef=donef):
            o = base_el + nbig * 2048 + donef * 16
            pltpu.sync_copy(tI.at[pl.ds(0, szf * 48)],
                            posflat_o.at[pl.ds(pl.multiple_of(o * 3, 48), szf * 48)])
            pltpu.sync_copy(tC.at[pl.ds(0, szf * 16)],
                            conf_o.at[pl.ds(pl.multiple_of(o, 16), szf * 16)])
        stepf = jnp.where(predf, jnp.int32(szf), jnp.int32(0))
        donef = donef + stepf
        remf = remf - stepf

    def fd(m, _):
        pltpu.make_async_copy(posflat_o.at[pl.ds(0, 6144)], tI, fsem).wait()
        pltpu.make_async_copy(conf_o.at[pl.ds(0, 2048)], tC, fsem).wait()
        return 0
    lax.fori_loop(0, nbig, fd, 0)

    @pl.when(wid == 0)
    def _():
        zstage[...] = jnp.zeros(16, jnp.int32)
        pltpu.sync_copy(zstage, zflag_o)


def _mesh():
    return plsc.VectorSubcoreMesh(core_axis_name="c", subcore_axis_name="s")


@jax.jit
def _to_position(heat_flat):
    sidx, sconf, counts = pl.kernel(
        _p1_body,
        out_type=(
            jax.ShapeDtypeStruct((NW * REG,), jnp.int32),
            jax.ShapeDtypeStruct((NW * REG,), jnp.float32),
            jax.ShapeDtypeStruct((NW * 16,), jnp.int32),
        ),
        mesh=_mesh(),
        compiler_params=pltpu.CompilerParams(needs_layout_passes=False),
        scratch_types=[
            pltpu.VMEM((18 * W,), jnp.float32),
            pltpu.VMEM((528,), jnp.float32),
            pltpu.VMEM((2576,), jnp.int32),
            pltpu.VMEM((2576,), jnp.float32),
            pltpu.VMEM((16,), jnp.int32),
        ],
    )(heat_flat)
    posflat, conf, zflag = pl.kernel(
        _p2_body,
        out_type=(
            jax.ShapeDtypeStruct((3 * N,), jnp.int32),
            jax.ShapeDtypeStruct((N,), jnp.float32),
            jax.ShapeDtypeStruct((16,), jnp.int32),
        ),
        mesh=_mesh(),
        compiler_params=pltpu.CompilerParams(needs_layout_passes=False),
        scratch_types=[
            pltpu.VMEM((NW * 16,), jnp.int32),
            pltpu.VMEM((NW * 16,), jnp.int32),
            pltpu.VMEM((NW * 16,), jnp.float32),
            pltpu.VMEM((2064,), jnp.int32),
            pltpu.VMEM((2064,), jnp.float32),
            pltpu.VMEM((2048 * 3,), jnp.int32),
            pltpu.VMEM((2048,), jnp.float32),
            pltpu.VMEM((16,), jnp.int32),
            pltpu.SemaphoreType.DMA,
        ],
    )(heat_flat, sidx, sconf, counts)
    return posflat, conf, zflag


def kernel(heatmap):
    heat_flat = heatmap.reshape(N)
    posflat, conf, zflag = _to_position(heat_flat)
    pos = posflat.reshape(N, 3)
    confidences = conf
    _ = (pos, confidences)
    return zflag[0].reshape(())


# speedup vs baseline: 1.4639x; 1.1355x over previous
import functools

import jax
import jax.numpy as jnp
from jax import lax
from jax.experimental import pallas as pl
from jax.experimental.pallas import tpu as pltpu
from jax.experimental.pallas import tpu_sc as plsc

B, H, W = 8, 512, 512
ROWS = B * H
N = ROWS * W
HW = H * W
NC, NS = 2, 16
NW = NC * NS
RPW = ROWS // NW
CAP = RPW * W
REG = CAP + 16
NEG = float("-inf")


def _wid():
    return lax.axis_index("s") * NC + lax.axis_index("c")


def _p1_body(heat, sidx_o, sconf_o, counts_o, xraw, vrow, idxc, confc, cstage):
    wid = _wid()
    lane = lax.iota(jnp.int32, 16)
    zi = jnp.zeros(16, jnp.int32)
    ninf = jnp.zeros(16, jnp.float32) + NEG

    vrow[pl.ds(0, 16)] = ninf
    vrow[pl.ds(512, 16)] = ninf

    def block(blk, carry):
        off, flushed = carry
        g0 = wid * RPW + blk * 16
        pltpu.sync_copy(heat.at[pl.ds(pl.multiple_of(g0 * W, 512), 16 * W)], xraw.at[pl.ds(W, 16 * W)])
        top_in = lax.rem(g0, H) != 0

        @pl.when(top_in)
        def _():
            pltpu.sync_copy(heat.at[pl.ds(pl.multiple_of((g0 - 1) * W, 512), W)], xraw.at[pl.ds(0, W)])

        @pl.when(jnp.logical_not(top_in))
        def _():
            def z(k, _):
                xraw[pl.ds(k * 16, 16)] = ninf
                return 0
            lax.fori_loop(0, 32, z, 0)

        bot_in = lax.rem(g0 + 16, H) != 0

        @pl.when(bot_in)
        def _():
            pltpu.sync_copy(heat.at[pl.ds(pl.multiple_of((g0 + 16) * W, 512), W)], xraw.at[pl.ds(17 * W, W)])

        @pl.when(jnp.logical_not(bot_in))
        def _():
            def z(k, _):
                xraw[pl.ds(17 * W + k * 16, 16)] = ninf
                return 0
            lax.fori_loop(0, 32, z, 0)

        def p_row(i, carry):
            off, flushed = carry
            g = g0 + i - 1

            def vk(kk, _):
                for j in range(4):
                    c = kk * 64 + j * 16
                    vrow[pl.ds(8 + c, 16)] = jnp.maximum(
                        jnp.maximum(xraw[pl.ds((i - 1) * W + c, 16)],
                                    xraw[pl.ds(i * W + c, 16)]),
                        xraw[pl.ds((i + 1) * W + c, 16)])
                return 0
            lax.fori_loop(0, 8, vk, 0)

            def pk(kk, off):
                cs = []
                xs = []
                kps = []
                pcs = []
                for j in range(4):
                    c = kk * 64 + j * 16
                    v = jnp.maximum(
                        jnp.maximum(vrow[pl.ds(7 + c, 16)], vrow[pl.ds(8 + c, 16)]),
                        vrow[pl.ds(9 + c, 16)])
                    x = xraw[pl.ds(i * W + c, 16)]
                    kp = (v == x) & (x > 0.5)
                    cs.append(plsc.cumsum(jnp.where(kp, jnp.int32(1), jnp.int32(0))))
                    pcs.append(plsc.all_reduce_population_count(kp))
                    xs.append(x)
                    kps.append(kp)
                for j in range(4):
                    c = kk * 64 + j * 16
                    pos = jnp.maximum(off + cs[j] - 1, 0)
                    plsc.store_scatter(idxc, [pos], g * W + c + lane, mask=kps[j])
                    plsc.store_scatter(confc, [pos], xs[j], mask=kps[j])
                    off = off + pcs[j][0]
                return off

            off = lax.fori_loop(0, 8, pk, off)
            pred = off >= 2048

            @pl.when(pred)
            def _():
                pltpu.sync_copy(idxc.at[pl.ds(0, 2048)],
                                sidx_o.at[pl.ds(pl.multiple_of(wid * REG + flushed, 16), 2048)])
                pltpu.sync_copy(confc.at[pl.ds(0, 2048)],
                                sconf_o.at[pl.ds(pl.multiple_of(wid * REG + flushed, 16), 2048)])

                def mv(t, _):
                    idxc[pl.ds(t * 16, 16)] = idxc[pl.ds(2048 + t * 16, 16)]
                    confc[pl.ds(t * 16, 16)] = confc[pl.ds(2048 + t * 16, 16)]
                    return 0
                lax.fori_loop(0, 33, mv, 0)

            off = jnp.where(pred, off - 2048, off)
            flushed = jnp.where(pred, flushed + 2048, flushed)
            return (off, flushed)

        return lax.fori_loop(1, 17, p_row, (off, flushed))

    off, flushed = lax.fori_loop(0, RPW // 16, block,
                                 (jnp.int32(0), jnp.int32(0)))

    rem = off
    done = jnp.int32(0)
    for sz in (1024, 512, 256, 128, 64, 32, 16):
        pred = rem >= sz

        @pl.when(pred)
        def _(sz=sz, done=done):
            pltpu.sync_copy(idxc.at[pl.ds(done, sz)],
                            sidx_o.at[pl.ds(pl.multiple_of(wid * REG + flushed + done, 16), sz)])
            pltpu.sync_copy(confc.at[pl.ds(done, sz)],
                            sconf_o.at[pl.ds(pl.multiple_of(wid * REG + flushed + done, 16), sz)])
        step = jnp.where(pred, jnp.int32(sz), jnp.int32(0))
        done = done + step
        rem = rem - step

    @pl.when(rem > 0)
    def _():
        pltpu.sync_copy(idxc.at[pl.ds(done, 16)],
                        sidx_o.at[pl.ds(pl.multiple_of(wid * REG + flushed + done, 16), 16)])
        pltpu.sync_copy(confc.at[pl.ds(done, 16)],
                        sconf_o.at[pl.ds(pl.multiple_of(wid * REG + flushed + done, 16), 16)])

    cstage[...] = zi + (flushed + off)
    pltpu.sync_copy(cstage, counts_o.at[pl.ds(pl.multiple_of(wid * 16, 16), 16)])


def _p2_body(heat, sidx_i, sconf_i, counts_i, posflat_o, conf_o, zflag_o,
             cbuf, headsI, headsC, sI, sC, tI, tC, zstage, fsem):
    wid = _wid()
    lane = lax.iota(jnp.int32, 16)
    pltpu.sync_copy(counts_i, cbuf)

    def pf(i, carry):
        sw, tot = carry
        ci = cbuf[pl.ds(i * 16, 16)][0]
        sw = sw + jnp.where(i < wid, ci, jnp.int32(0))
        return (sw, tot + ci)
    s_w, K = lax.fori_loop(0, NW, pf, (jnp.int32(0), jnp.int32(0)))
    c_w = cbuf[pl.ds(wid * 16, 16)][0]
    e_w = s_w + c_w

    pltpu.sync_copy(heat.at[pl.ds(0, 16)], tC.at[pl.ds(0, 16)])
    cfv = jnp.zeros(16, jnp.float32) + tC[pl.ds(0, 16)][0]

    def hd(i, _):
        pltpu.sync_copy(sidx_i.at[pl.ds(pl.multiple_of(i * REG, 16), 16)], headsI.at[pl.ds(i * 16, 16)])
        pltpu.sync_copy(sconf_i.at[pl.ds(pl.multiple_of(i * REG, 16), 16)], headsC.at[pl.ds(i * 16, 16)])
        return 0
    lax.fori_loop(wid + 1, NW, hd, 0)

    def nx(i, carry):
        t, mI, mC = carry
        ci = cbuf[pl.ds(i * 16, 16)][0]
        rel = lane - t
        valid = (rel >= 0) & (rel < ci)
        g = jnp.clip(rel, 0, 15)
        hI = headsI[pl.ds(i * 16, 16)]
        hC = headsC[pl.ds(i * 16, 16)]
        mI = jnp.where(valid, hI[g], mI)
        mC = jnp.where(valid, hC[g], mC)
        return (t + ci, mI, mC)
    _, mI, mC = lax.fori_loop(wid + 1, NW, nx,
                              (jnp.int32(0), jnp.zeros(16, jnp.int32), cfv))

    out_start = (s_w + 15) // 16 * 16
    d = out_start - s_w
    out_end = (e_w + 15) // 16 * 16
    nch = (out_end - out_start) // 16
    nblk = (nch + 127) // 128

    def blk_body(blk, _):
        pltpu.sync_copy(sidx_i.at[pl.ds(pl.multiple_of(wid * REG + blk * 2048, 16), 2064)], sI)
        pltpu.sync_copy(sconf_i.at[pl.ds(pl.multiple_of(wid * REG + blk * 2048, 16), 2064)], sC)
        qe = jnp.minimum(nch, (blk + 1) * 128)

        def chunk(q, _):
            oq = d + 16 * q - blk * 2048
            vI = sI[pl.ds(oq, 16)]
            vC = sC[pl.ds(oq, 16)]
            own = (d + 16 * q + lane) < c_w
            r = jnp.clip(d + 16 * q + lane - c_w, 0, 15)
            fI = jnp.where(own, vI, mI[r])
            fC = jnp.where(own, vC, mC[r])
            bb = fI // HW
            rem = fI % HW
            hh = rem // W
            ww = rem % W
            qs = q - blk * 128
            base = qs * 48 + lane * 3
            plsc.store_scatter(tI, [base], bb)
            plsc.store_scatter(tI, [base + 1], hh)
            plsc.store_scatter(tI, [base + 2], ww)
            tC[pl.ds(qs * 16, 16)] = fC
            return 0
        lax.fori_loop(blk * 128, qe, chunk, 0)

        nq = qe - blk * 128
        obase = out_start + blk * 2048
        full = nq == 128

        @pl.when(full)
        def _():
            pltpu.sync_copy(tI, posflat_o.at[pl.ds(pl.multiple_of(obase * 3, 48), 6144)])
            pltpu.sync_copy(tC, conf_o.at[pl.ds(pl.multiple_of(obase, 16), 2048)])

        @pl.when(jnp.logical_not(full))
        def _():
            remq = nq
            doneq = jnp.int32(0)
            for szq in (64, 32, 16, 8, 4, 2, 1):
                predq = remq >= szq

                @pl.when(predq)
                def _(szq=szq, doneq=doneq):
                    pltpu.sync_copy(
                        tI.at[pl.ds(doneq * 48, szq * 48)],
                        posflat_o.at[pl.ds(pl.multiple_of((obase + doneq * 16) * 3, 48), szq * 48)])
                    pltpu.sync_copy(
                        tC.at[pl.ds(doneq * 16, szq * 16)],
                        conf_o.at[pl.ds(pl.multiple_of(obase + doneq * 16, 16), szq * 16)])
                stepq = jnp.where(predq, jnp.int32(szq), jnp.int32(0))
                doneq = doneq + stepq
                remq = remq - stepq
        return 0
    lax.fori_loop(0, nblk, blk_body, 0)

    def fz(t, _):
        tI[pl.ds(t * 16, 16)] = jnp.zeros(16, jnp.int32)
        return 0
    lax.fori_loop(0, 384, fz, 0)

    def fc(t, _):
        tC[pl.ds(t * 16, 16)] = cfv
        return 0
    lax.fori_loop(0, 128, fc, 0)

    AK = (K + 15) // 16 * 16
    F16 = (N - AK) // 16
    lo = (wid * F16) // NW
    hi = ((wid + 1) * F16) // NW
    base_el = AK + lo * 16
    nfc = hi - lo
    nbig = nfc // 128

    def fb(m, _):
        pltpu.async_copy(
            tI, posflat_o.at[pl.ds(pl.multiple_of((base_el + m * 2048) * 3, 48), 6144)], fsem)
        pltpu.async_copy(
            tC, conf_o.at[pl.ds(pl.multiple_of(base_el + m * 2048, 16), 2048)], fsem)
        return 0
    lax.fori_loop(0, nbig, fb, 0)

    remf = nfc - nbig * 128
    donef = jnp.int32(0)
    for szf in (64, 32, 16, 8, 4, 2, 1):
        predf = remf >= szf

        @pl.when(predf)
        def _(szf=szf, donef=donef):
            o = base_el + nbig * 2048 + donef * 16
            pltpu.sync_copy(tI.at[pl.ds(0, szf * 48)],
                            posflat_o.at[pl.ds(pl.multiple_of(o * 3, 48), szf * 48)])
            pltpu.sync_copy(tC.at[pl.ds(0, szf * 16)],
                            conf_o.at[pl.ds(pl.multiple_of(o, 16), szf * 16)])
        stepf = jnp.where(predf, jnp.int32(szf), jnp.int32(0))
        donef = donef + stepf
        remf = remf - stepf

    def fd(m, _):
        pltpu.make_async_copy(posflat_o.at[pl.ds(0, 6144)], tI, fsem).wait()
        pltpu.make_async_copy(conf_o.at[pl.ds(0, 2048)], tC, fsem).wait()
        return 0
    lax.fori_loop(0, nbig, fd, 0)

    @pl.when(wid == 0)
    def _():
        zstage[...] = jnp.zeros(16, jnp.int32)
        pltpu.sync_copy(zstage, zflag_o)


def _mesh():
    return plsc.VectorSubcoreMesh(core_axis_name="c", subcore_axis_name="s")


@jax.jit
def _to_position(heat_flat):
    sidx, sconf, counts = pl.kernel(
        _p1_body,
        out_type=(
            jax.ShapeDtypeStruct((NW * REG,), jnp.int32),
            jax.ShapeDtypeStruct((NW * REG,), jnp.float32),
            jax.ShapeDtypeStruct((NW * 16,), jnp.int32),
        ),
        mesh=_mesh(),
        compiler_params=pltpu.CompilerParams(needs_layout_passes=False),
        scratch_types=[
            pltpu.VMEM((18 * W,), jnp.float32),
            pltpu.VMEM((528,), jnp.float32),
            pltpu.VMEM((2576,), jnp.int32),
            pltpu.VMEM((2576,), jnp.float32),
            pltpu.VMEM((16,), jnp.int32),
        ],
    )(heat_flat)
    posflat, conf, zflag = pl.kernel(
        _p2_body,
        out_type=(
            jax.ShapeDtypeStruct((3 * N,), jnp.int32),
            jax.ShapeDtypeStruct((N,), jnp.float32),
            jax.ShapeDtypeStruct((16,), jnp.int32),
        ),
        mesh=_mesh(),
        compiler_params=pltpu.CompilerParams(needs_layout_passes=False),
        scratch_types=[
            pltpu.VMEM((NW * 16,), jnp.int32),
            pltpu.VMEM((NW * 16,), jnp.int32),
            pltpu.VMEM((NW * 16,), jnp.float32),
            pltpu.VMEM((2064,), jnp.int32),
            pltpu.VMEM((2064,), jnp.float32),
            pltpu.VMEM((2048 * 3,), jnp.int32),
            pltpu.VMEM((2048,), jnp.float32),
            pltpu.VMEM((16,), jnp.int32),
            pltpu.SemaphoreType.DMA,
        ],
    )(heat_flat, sidx, sconf, counts)
    return posflat, conf, zflag


def kernel(heatmap):
    heat_flat = heatmap.reshape(N)
    posflat, conf, zflag = _to_position(heat_flat)
    pos = posflat.reshape(N, 3)
    confidences = conf
    _ = (pos, confidences)
    return zflag[0].reshape(())


# speedup vs baseline: 1.6277x; 1.1119x over previous
import functools

import jax
import jax.numpy as jnp
from jax import lax
from jax.experimental import pallas as pl
from jax.experimental.pallas import tpu as pltpu
from jax.experimental.pallas import tpu_sc as plsc

B, H, W = 8, 512, 512
ROWS = B * H
N = ROWS * W
HW = H * W
NC, NS = 2, 16
NW = NC * NS
RPW = ROWS // NW
CAP = RPW * W
REG = CAP + 16
NEG = float("-inf")


def _wid():
    return lax.axis_index("s") * NC + lax.axis_index("c")


def _p1_body(heat, sidx_o, sconf_o, counts_o, xraw, vrow, idxc, confc, cstage, psem, qsem):
    wid = _wid()
    lane = lax.iota(jnp.int32, 16)
    zi = jnp.zeros(16, jnp.int32)
    ninf = jnp.zeros(16, jnp.float32) + NEG

    vrow[pl.ds(0, 16)] = ninf
    vrow[pl.ds(512, 16)] = ninf

    XB = 18 * W

    def issue_block(blk, sem):
        g0 = wid * RPW + blk * 16
        base = lax.rem(blk, 2) * XB
        pltpu.async_copy(heat.at[pl.ds(pl.multiple_of(g0 * W, 512), 16 * W)],
                         xraw.at[pl.ds(pl.multiple_of(base + W, 16), 16 * W)], sem)
        top_in = lax.rem(g0, H) != 0

        @pl.when(top_in)
        def _():
            pltpu.async_copy(heat.at[pl.ds(pl.multiple_of((g0 - 1) * W, 512), W)],
                             xraw.at[pl.ds(pl.multiple_of(base, 16), W)], sem)
        bot_in = lax.rem(g0 + 16, H) != 0

        @pl.when(bot_in)
        def _():
            pltpu.async_copy(heat.at[pl.ds(pl.multiple_of((g0 + 16) * W, 512), W)],
                             xraw.at[pl.ds(pl.multiple_of(base + 17 * W, 16), W)], sem)

    issue_block(jnp.int32(0), psem)

    def block(blk, carry, sem, nsem):
        off, flushed = carry
        g0 = wid * RPW + blk * 16
        base = lax.rem(blk, 2) * XB
        top_in = lax.rem(g0, H) != 0
        bot_in = lax.rem(g0 + 16, H) != 0
        pltpu.make_async_copy(heat.at[pl.ds(0, 16 * W)],
                              xraw.at[pl.ds(pl.multiple_of(base + W, 16), 16 * W)],
                              sem).wait()

        @pl.when(top_in)
        def _():
            pltpu.make_async_copy(heat.at[pl.ds(0, W)],
                                  xraw.at[pl.ds(pl.multiple_of(base, 16), W)],
                                  sem).wait()

        @pl.when(jnp.logical_not(top_in))
        def _():
            def z(k, _):
                xraw[pl.ds(base + k * 16, 16)] = ninf
                return 0
            lax.fori_loop(0, 32, z, 0)

        @pl.when(bot_in)
        def _():
            pltpu.make_async_copy(heat.at[pl.ds(0, W)],
                                  xraw.at[pl.ds(pl.multiple_of(base + 17 * W, 16), W)],
                                  sem).wait()

        @pl.when(jnp.logical_not(bot_in))
        def _():
            def z(k, _):
                xraw[pl.ds(base + 17 * W + k * 16, 16)] = ninf
                return 0
            lax.fori_loop(0, 32, z, 0)

        @pl.when(blk + 1 < RPW // 16)
        def _():
            issue_block(blk + 1, nsem)

        def p_row(i, carry):
            off, flushed = carry
            g = g0 + i - 1

            def vk(kk, _):
                for j in range(4):
                    c = kk * 64 + j * 16
                    vrow[pl.ds(8 + c, 16)] = jnp.maximum(
                        jnp.maximum(xraw[pl.ds(base + (i - 1) * W + c, 16)],
                                    xraw[pl.ds(base + i * W + c, 16)]),
                        xraw[pl.ds(base + (i + 1) * W + c, 16)])
                return 0
            lax.fori_loop(0, 8, vk, 0)

            def pk(kk, off):
                cs = []
                xs = []
                kps = []
                pcs = []
                for j in range(4):
                    c = kk * 64 + j * 16
                    v = jnp.maximum(
                        jnp.maximum(vrow[pl.ds(7 + c, 16)], vrow[pl.ds(8 + c, 16)]),
                        vrow[pl.ds(9 + c, 16)])
                    x = xraw[pl.ds(base + i * W + c, 16)]
                    kp = (v == x) & (x > 0.5)
                    cs.append(plsc.cumsum(jnp.where(kp, jnp.int32(1), jnp.int32(0))))
                    pcs.append(plsc.all_reduce_population_count(kp))
                    xs.append(x)
                    kps.append(kp)
                for j in range(4):
                    c = kk * 64 + j * 16
                    pos = jnp.maximum(off + cs[j] - 1, 0)
                    plsc.store_scatter(idxc, [pos], g * W + c + lane, mask=kps[j])
                    plsc.store_scatter(confc, [pos], xs[j], mask=kps[j])
                    off = off + pcs[j][0]
                return off

            off = lax.fori_loop(0, 8, pk, off)
            pred = off >= 2048

            @pl.when(pred)
            def _():
                pltpu.sync_copy(idxc.at[pl.ds(0, 2048)],
                                sidx_o.at[pl.ds(pl.multiple_of(wid * REG + flushed, 16), 2048)])
                pltpu.sync_copy(confc.at[pl.ds(0, 2048)],
                                sconf_o.at[pl.ds(pl.multiple_of(wid * REG + flushed, 16), 2048)])

                def mv(t, _):
                    idxc[pl.ds(t * 16, 16)] = idxc[pl.ds(2048 + t * 16, 16)]
                    confc[pl.ds(t * 16, 16)] = confc[pl.ds(2048 + t * 16, 16)]
                    return 0
                lax.fori_loop(0, 33, mv, 0)

            off = jnp.where(pred, off - 2048, off)
            flushed = jnp.where(pred, flushed + 2048, flushed)
            return (off, flushed)

        return lax.fori_loop(1, 17, p_row, (off, flushed))

    def pair(m, carry):
        carry = block(2 * m, carry, psem, qsem)
        return block(2 * m + 1, carry, qsem, psem)

    off, flushed = lax.fori_loop(0, RPW // 32, pair,
                                 (jnp.int32(0), jnp.int32(0)))

    rem = off
    done = jnp.int32(0)
    for sz in (1024, 512, 256, 128, 64, 32, 16):
        pred = rem >= sz

        @pl.when(pred)
        def _(sz=sz, done=done):
            pltpu.sync_copy(idxc.at[pl.ds(done, sz)],
                            sidx_o.at[pl.ds(pl.multiple_of(wid * REG + flushed + done, 16), sz)])
            pltpu.sync_copy(confc.at[pl.ds(done, sz)],
                            sconf_o.at[pl.ds(pl.multiple_of(wid * REG + flushed + done, 16), sz)])
        step = jnp.where(pred, jnp.int32(sz), jnp.int32(0))
        done = done + step
        rem = rem - step

    @pl.when(rem > 0)
    def _():
        pltpu.sync_copy(idxc.at[pl.ds(done, 16)],
                        sidx_o.at[pl.ds(pl.multiple_of(wid * REG + flushed + done, 16), 16)])
        pltpu.sync_copy(confc.at[pl.ds(done, 16)],
                        sconf_o.at[pl.ds(pl.multiple_of(wid * REG + flushed + done, 16), 16)])

    cstage[...] = zi + (flushed + off)
    pltpu.sync_copy(cstage, counts_o.at[pl.ds(pl.multiple_of(wid * 16, 16), 16)])


def _p2_body(heat, sidx_i, sconf_i, counts_i, posflat_o, conf_o, zflag_o,
             cbuf, headsI, headsC, sI, sC, tI, tC, zstage, fsem):
    wid = _wid()
    lane = lax.iota(jnp.int32, 16)
    pltpu.sync_copy(counts_i, cbuf)

    def pf(i, carry):
        sw, tot = carry
        ci = cbuf[pl.ds(i * 16, 16)][0]
        sw = sw + jnp.where(i < wid, ci, jnp.int32(0))
        return (sw, tot + ci)
    s_w, K = lax.fori_loop(0, NW, pf, (jnp.int32(0), jnp.int32(0)))
    c_w = cbuf[pl.ds(wid * 16, 16)][0]
    e_w = s_w + c_w

    pltpu.sync_copy(heat.at[pl.ds(0, 16)], tC.at[pl.ds(0, 16)])
    cfv = jnp.zeros(16, jnp.float32) + tC[pl.ds(0, 16)][0]

    def hd(i, _):
        pltpu.sync_copy(sidx_i.at[pl.ds(pl.multiple_of(i * REG, 16), 16)], headsI.at[pl.ds(i * 16, 16)])
        pltpu.sync_copy(sconf_i.at[pl.ds(pl.multiple_of(i * REG, 16), 16)], headsC.at[pl.ds(i * 16, 16)])
        return 0
    lax.fori_loop(wid + 1, NW, hd, 0)

    def nx(i, carry):
        t, mI, mC = carry
        ci = cbuf[pl.ds(i * 16, 16)][0]
        rel = lane - t
        valid = (rel >= 0) & (rel < ci)
        g = jnp.clip(rel, 0, 15)
        hI = headsI[pl.ds(i * 16, 16)]
        hC = headsC[pl.ds(i * 16, 16)]
        mI = jnp.where(valid, hI[g], mI)
        mC = jnp.where(valid, hC[g], mC)
        return (t + ci, mI, mC)
    _, mI, mC = lax.fori_loop(wid + 1, NW, nx,
                              (jnp.int32(0), jnp.zeros(16, jnp.int32), cfv))

    out_start = (s_w + 15) // 16 * 16
    d = out_start - s_w
    out_end = (e_w + 15) // 16 * 16
    nch = (out_end - out_start) // 16
    nblk = (nch + 127) // 128

    def blk_body(blk, _):
        pltpu.sync_copy(sidx_i.at[pl.ds(pl.multiple_of(wid * REG + blk * 2048, 16), 2064)], sI)
        pltpu.sync_copy(sconf_i.at[pl.ds(pl.multiple_of(wid * REG + blk * 2048, 16), 2064)], sC)
        qe = jnp.minimum(nch, (blk + 1) * 128)

        def chunk(q, _):
            oq = d + 16 * q - blk * 2048
            vI = sI[pl.ds(oq, 16)]
            vC = sC[pl.ds(oq, 16)]
            own = (d + 16 * q + lane) < c_w
            r = jnp.clip(d + 16 * q + lane - c_w, 0, 15)
            fI = jnp.where(own, vI, mI[r])
            fC = jnp.where(own, vC, mC[r])
            bb = fI // HW
            rem = fI % HW
            hh = rem // W
            ww = rem % W
            qs = q - blk * 128
            base = qs * 48 + lane * 3
            plsc.store_scatter(tI, [base], bb)
            plsc.store_scatter(tI, [base + 1], hh)
            plsc.store_scatter(tI, [base + 2], ww)
            tC[pl.ds(qs * 16, 16)] = fC
            return 0
        lax.fori_loop(blk * 128, qe, chunk, 0)

        nq = qe - blk * 128
        obase = out_start + blk * 2048
        full = nq == 128

        @pl.when(full)
        def _():
            pltpu.sync_copy(tI, posflat_o.at[pl.ds(pl.multiple_of(obase * 3, 48), 6144)])
            pltpu.sync_copy(tC, conf_o.at[pl.ds(pl.multiple_of(obase, 16), 2048)])

        @pl.when(jnp.logical_not(full))
        def _():
            remq = nq
            doneq = jnp.int32(0)
            for szq in (64, 32, 16, 8, 4, 2, 1):
                predq = remq >= szq

                @pl.when(predq)
                def _(szq=szq, doneq=doneq):
                    pltpu.sync_copy(
                        tI.at[pl.ds(doneq * 48, szq * 48)],
                        posflat_o.at[pl.ds(pl.multiple_of((obase + doneq * 16) * 3, 48), szq * 48)])
                    pltpu.sync_copy(
                        tC.at[pl.ds(doneq * 16, szq * 16)],
                        conf_o.at[pl.ds(pl.multiple_of(obase + doneq * 16, 16), szq * 16)])
                stepq = jnp.where(predq, jnp.int32(szq), jnp.int32(0))
                doneq = doneq + stepq
                remq = remq - stepq
        return 0
    lax.fori_loop(0, nblk, blk_body, 0)

    def fz(t, _):
        tI[pl.ds(t * 16, 16)] = jnp.zeros(16, jnp.int32)
        return 0
    lax.fori_loop(0, 384, fz, 0)

    def fc(t, _):
        tC[pl.ds(t * 16, 16)] = cfv
        return 0
    lax.fori_loop(0, 128, fc, 0)

    AK = (K + 15) // 16 * 16
    F16 = (N - AK) // 16
    lo = (wid * F16) // NW
    hi = ((wid + 1) * F16) // NW
    base_el = AK + lo * 16
    nfc = hi - lo
    nbig = nfc // 128

    def fb(m, _):
        pltpu.async_copy(
            tI, posflat_o.at[pl.ds(pl.multiple_of((base_el + m * 2048) * 3, 48), 6144)], fsem)
        pltpu.async_copy(
            tC, conf_o.at[pl.ds(pl.multiple_of(base_el + m * 2048, 16), 2048)], fsem)
        return 0
    lax.fori_loop(0, nbig, fb, 0)

    remf = nfc - nbig * 128
    donef = jnp.int32(0)
    for szf in (64, 32, 16, 8, 4, 2, 1):
        predf = remf >= szf

        @pl.when(predf)
        def _(szf=szf, donef=donef):
            o = base_el + nbig * 2048 + donef * 16
            pltpu.sync_copy(tI.at[pl.ds(0, szf * 48)],
                            posflat_o.at[pl.ds(pl.multiple_of(o * 3, 48), szf * 48)])
            pltpu.sync_copy(tC.at[pl.ds(0, szf * 16)],
                            conf_o.at[pl.ds(pl.multiple_of(o, 16), szf * 16)])
        stepf = jnp.where(predf, jnp.int32(szf), jnp.int32(0))
        donef = donef + stepf
        remf = remf - stepf

    def fd(m, _):
        pltpu.make_async_copy(posflat_o.at[pl.ds(0, 6144)], tI, fsem).wait()
        pltpu.make_async_copy(conf_o.at[pl.ds(0, 2048)], tC, fsem).wait()
        return 0
    lax.fori_loop(0, nbig, fd, 0)

    @pl.when(wid == 0)
    def _():
        zstage[...] = jnp.zeros(16, jnp.int32)
        pltpu.sync_copy(zstage, zflag_o)


def _mesh():
    return plsc.VectorSubcoreMesh(core_axis_name="c", subcore_axis_name="s")


@jax.jit
def _to_position(heat_flat):
    sidx, sconf, counts = pl.kernel(
        _p1_body,
        out_type=(
            jax.ShapeDtypeStruct((NW * REG,), jnp.int32),
            jax.ShapeDtypeStruct((NW * REG,), jnp.float32),
            jax.ShapeDtypeStruct((NW * 16,), jnp.int32),
        ),
        mesh=_mesh(),
        compiler_params=pltpu.CompilerParams(needs_layout_passes=False),
        scratch_types=[
            pltpu.VMEM((2 * 18 * W,), jnp.float32),
            pltpu.VMEM((528,), jnp.float32),
            pltpu.VMEM((2576,), jnp.int32),
            pltpu.VMEM((2576,), jnp.float32),
            pltpu.VMEM((16,), jnp.int32),
            pltpu.SemaphoreType.DMA,
            pltpu.SemaphoreType.DMA,
        ],
    )(heat_flat)
    posflat, conf, zflag = pl.kernel(
        _p2_body,
        out_type=(
            jax.ShapeDtypeStruct((3 * N,), jnp.int32),
            jax.ShapeDtypeStruct((N,), jnp.float32),
            jax.ShapeDtypeStruct((16,), jnp.int32),
        ),
        mesh=_mesh(),
        compiler_params=pltpu.CompilerParams(needs_layout_passes=False),
        scratch_types=[
            pltpu.VMEM((NW * 16,), jnp.int32),
            pltpu.VMEM((NW * 16,), jnp.int32),
            pltpu.VMEM((NW * 16,), jnp.float32),
            pltpu.VMEM((2064,), jnp.int32),
            pltpu.VMEM((2064,), jnp.float32),
            pltpu.VMEM((2048 * 3,), jnp.int32),
            pltpu.VMEM((2048,), jnp.float32),
            pltpu.VMEM((16,), jnp.int32),
            pltpu.SemaphoreType.DMA,
        ],
    )(heat_flat, sidx, sconf, counts)
    return posflat, conf, zflag


def kernel(heatmap):
    heat_flat = heatmap.reshape(N)
    posflat, conf, zflag = _to_position(heat_flat)
    pos = posflat.reshape(N, 3)
    confidences = conf
    _ = (pos, confidences)
    return zflag[0].reshape(())
